# baseline (speedup 1.0000x reference)
import functools

import jax
import jax.numpy as jnp
from jax import lax
from jax.experimental import pallas as pl
from jax.experimental.pallas import tpu as pltpu
from jax.experimental.pallas import tpu_sc as plsc

_NUM_ATOMS = 262144
_TABLE_ROWS = 128
_OUT_DIM = 128

_info = plsc.get_sparse_core_info()
_NC = _info.num_cores
_NS = _info.num_subcores
_NW = _NC * _NS
_B_PER_W = _NUM_ATOMS // _NW
_CHUNK = 128
_N_CHUNKS = _B_PER_W // _CHUNK
_G = 2
_SG = _G * _CHUNK
_N_SG = _B_PER_W // _SG
_NBUF = 3


def _project_body(fm_ref, w_ref, b_ref, out_ref):
    out_ref[...] = lax.dot_general(
        fm_ref[...], w_ref[...], (((1,), (1,)), ((), ())),
        preferred_element_type=jnp.float32) + b_ref[...]


def _project(feature_map, W, b):
    return pl.pallas_call(
        _project_body,
        out_shape=jax.ShapeDtypeStruct((_TABLE_ROWS, _OUT_DIM), jnp.float32),
    )(feature_map, W, b.reshape(1, _OUT_DIM))


_mesh = plsc.VectorSubcoreMesh(core_axis_name="c", subcore_axis_name="s")


@functools.partial(
    pl.kernel,
    mesh=_mesh,
    out_type=jax.ShapeDtypeStruct((_NUM_ATOMS, _OUT_DIM), jnp.float32),
    scratch_types=[
        pltpu.VMEM_SHARED((_TABLE_ROWS, _OUT_DIM), jnp.float32),
        pltpu.VMEM((_N_CHUNKS, _CHUNK), jnp.int32),
        pltpu.VMEM((_NBUF, _SG, _OUT_DIM), jnp.float32),
    ]
    + [pltpu.SemaphoreType.DMA] * (2 * _NBUF),
)
def _gather(table_hbm, idx_hbm, out_hbm, table_s, idx_v, rows_v, *sems):
    gsems = sems[:_NBUF]
    wsems = sems[_NBUF:]
    sid = lax.axis_index("s")
    wid = sid * _NC + lax.axis_index("c")
    base = wid * _B_PER_W

    idx_copy = pltpu.async_copy(idx_hbm.at[wid], idx_v, wsems[0])

    @pl.when(sid == 0)
    def _stage():
        pltpu.sync_copy(table_hbm, table_s)

    plsc.subcore_barrier()
    idx_copy.wait()

    def group(p, q, wait_write):
        if wait_write:
            pltpu.make_async_copy(
                rows_v.at[q], out_hbm.at[pl.ds(base, _SG)], wsems[q]).wait()
        handles = [
            pltpu.async_copy(
                table_s.at[idx_v.at[p * _G + k]],
                rows_v.at[q, pl.ds(k * _CHUNK, _CHUNK)],
                gsems[q])
            for k in range(_G)
        ]
        for h in handles:
            h.wait()
        pltpu.async_copy(
            rows_v.at[q], out_hbm.at[pl.ds(base + p * _SG, _SG)], wsems[q])

    for p in range(_NBUF):
        group(p, p, wait_write=False)

    def body(gg, carry):
        for s in range(_NBUF):
            group(_NBUF * gg + s, s, wait_write=True)
        return carry

    lax.fori_loop(1, (_N_SG - 2) // _NBUF, body, 0)

    for p in range(_N_SG - 2, _N_SG):
        group(p, p % _NBUF, wait_write=True)

    for q in range(_NBUF):
        pltpu.make_async_copy(
            rows_v.at[q], out_hbm.at[pl.ds(base, _SG)], wsems[q]).wait()


def kernel(src, feature_map, W, b):
    proj = _project(feature_map, W, b)
    idx = src.astype(jnp.int32).reshape(_NW, _N_CHUNKS, _CHUNK)
    return _gather(proj, idx)
